# baseline (speedup 1.0000x reference)
import functools

import jax
import jax.numpy as jnp
from jax import lax
from jax.experimental import pallas as pl
from jax.experimental.pallas import tpu as pltpu
from jax.experimental.pallas import tpu_sc as plsc

NC = 2
NS = 16
L = 16
NW = NC * NS

B = 16384
F = 26
D = 16
BW = B // NW
NBLK = BW // 128
TILE = 8 * 128
WTW = NBLK * TILE


def _sc_embed(table, ids3, valT, tabT):
    mesh = plsc.VectorSubcoreMesh(core_axis_name="c", subcore_axis_name="s")

    @functools.partial(
        pl.kernel,
        out_type=jax.ShapeDtypeStruct((F, 2, (B // 128) * TILE), jnp.float32),
        mesh=mesh,
        compiler_params=pltpu.CompilerParams(
            use_tc_tiling_on_sc=False, needs_layout_passes=False
        ),
        scratch_types=[
            pltpu.VMEM((F, NBLK, 128), jnp.int32),
            pltpu.VMEM((F, BW), jnp.float32),
            pltpu.VMEM((2, BW, D), jnp.float32),
            pltpu.VMEM((2, 2, WTW), jnp.float32),
            pltpu.SemaphoreType.DMA,
            pltpu.SemaphoreType.DMA,
        ],
    )
    def body(tab_hbm, ids_hbm, val_hbm, tabT_hbm, out_hbm,
             idx_v, val_v, rows_v, stage_v, sem_g, sem_o):
        wid = lax.axis_index("s") * NC + lax.axis_index("c")
        b0 = wid * BW
        pltpu.sync_copy(ids_hbm.at[:, pl.ds(wid * NBLK, NBLK), :], idx_v)
        pltpu.sync_copy(val_hbm.at[:, pl.ds(b0, BW)], val_v)

        lanes = lax.iota(jnp.int32, L)

        def fire_gathers(f, p):
            return [
                pltpu.async_copy(
                    tab_hbm.at[idx_v.at[f, blk]],
                    rows_v.at[p, pl.ds(blk * 128, 128)],
                    sem_g,
                )
                for blk in range(NBLK)
            ]

        def compute_field(f, p):
            pconst = jnp.full((L,), p, jnp.int32)

            def grp(g, carry):
                valv = val_v[f, pl.ds(g * L, L)]
                ridx = lanes + g * L
                off = (g // 8) * TILE + (g % 8) * L
                for e in range(D):
                    vec = plsc.load_gather(
                        rows_v, [pconst, ridx, jnp.full((L,), e, jnp.int32)]
                    )
                    stage_v[p, e // 8, pl.ds(off + (e % 8) * 128, L)] = vec * valv
                return carry

            lax.fori_loop(0, BW // L, grp, 0)

        gacc = {0: fire_gathers(0, 0)}
        oacc = {}
        for f in range(F):
            p = f % 2
            if f + 1 < F:
                gacc[f + 1] = fire_gathers(f + 1, (f + 1) % 2)
            for cp in gacc[f]:
                cp.wait()
            compute_field(f, p)
            if f >= 2:
                for cp in oacc[f - 2]:
                    cp.wait()
            oacc[f] = [
                pltpu.async_copy(
                    stage_v.at[p, s],
                    out_hbm.at[f, s, pl.ds(wid * WTW, WTW)],
                    sem_o,
                )
                for s in range(2)
            ]
        for f in (F - 2, F - 1):
            for cp in oacc[f]:
                cp.wait()

    return body(table, ids3, valT, tabT)


def kernel(ids, value, table):
    ids3 = ids.T.astype(jnp.int32).reshape(F, B // 128, 128)
    valT = value.T
    out = _sc_embed(table, ids3, valT, table.T)
    out5 = out.reshape(F, 2, B // 128, 8, 128)
    return out5.transpose(2, 4, 0, 1, 3).reshape(B, F, D)


# speedup vs baseline: 3.2006x; 3.2006x over previous
import functools

import jax
import jax.numpy as jnp
from jax import lax
from jax.experimental import pallas as pl
from jax.experimental.pallas import tpu as pltpu
from jax.experimental.pallas import tpu_sc as plsc

NC = 2
NS = 16
L = 16
NW = NC * NS

B = 16384
F = 26
D = 16
BW = B // NW
NBLK = BW // 128
TILE = 8 * 128
WTW = NBLK * TILE


def _sc_embed(table, ids3, valT):
    mesh = plsc.VectorSubcoreMesh(core_axis_name="c", subcore_axis_name="s")

    @functools.partial(
        pl.kernel,
        out_type=jax.ShapeDtypeStruct((F, 2, (B // 128) * TILE), jnp.float32),
        mesh=mesh,
        compiler_params=pltpu.CompilerParams(
            use_tc_tiling_on_sc=False, needs_layout_passes=False
        ),
        scratch_types=[
            pltpu.VMEM((F, NBLK, 128), jnp.int32),
            pltpu.VMEM((F, BW), jnp.float32),
            pltpu.VMEM((2, BW, D), jnp.float32),
            pltpu.VMEM((2, 2, WTW), jnp.float32),
            pltpu.SemaphoreType.DMA,
            pltpu.SemaphoreType.DMA,
        ],
    )
    def body(tab_hbm, ids_hbm, val_hbm, out_hbm,
             idx_v, val_v, rows_v, stage_v, sem_g, sem_o):
        wid = lax.axis_index("s") * NC + lax.axis_index("c")
        b0 = wid * BW
        pltpu.sync_copy(ids_hbm.at[:, pl.ds(wid * NBLK, NBLK), :], idx_v)
        pltpu.sync_copy(val_hbm.at[:, pl.ds(b0, BW)], val_v)

        lanes = lax.iota(jnp.int32, L)

        def fire_gathers(f, p):
            return [
                pltpu.async_copy(
                    tab_hbm.at[idx_v.at[f, blk]],
                    rows_v.at[p, pl.ds(blk * 128, 128)],
                    sem_g,
                )
                for blk in range(NBLK)
            ]

        def compute_field(f, p):
            pconst = jnp.full((L,), p, jnp.int32)

            def grp(g, carry):
                valv = val_v[f, pl.ds(g * L, L)]
                ridx = lanes + g * L
                off = (g // 8) * TILE + (g % 8) * L
                for j in range(D):
                    e_vec = (lanes + j) & (D - 1)
                    vec = plsc.load_gather(rows_v, [pconst, ridx, e_vec])
                    plsc.store_scatter(
                        stage_v,
                        [pconst, e_vec >> 3, off + (e_vec & 7) * 128 + lanes],
                        vec * valv,
                    )
                return carry

            lax.fori_loop(0, BW // L, grp, 0)

        gacc = {0: fire_gathers(0, 0)}
        oacc = {}
        for f in range(F):
            p = f % 2
            if f + 1 < F:
                gacc[f + 1] = fire_gathers(f + 1, (f + 1) % 2)
            for cp in gacc[f]:
                cp.wait()
            compute_field(f, p)
            if f >= 2:
                for cp in oacc[f - 2]:
                    cp.wait()
            oacc[f] = [
                pltpu.async_copy(
                    stage_v.at[p, s],
                    out_hbm.at[f, s, pl.ds(wid * WTW, WTW)],
                    sem_o,
                )
                for s in range(2)
            ]
        for f in (F - 2, F - 1):
            for cp in oacc[f]:
                cp.wait()

    return body(table, ids3, valT)


def kernel(ids, value, table):
    ids3 = ids.T.astype(jnp.int32).reshape(F, B // 128, 128)
    valT = value.T
    out = _sc_embed(table, ids3, valT)
    out5 = out.reshape(F, 2, B // 128, 8, 128)
    return out5.transpose(2, 4, 0, 1, 3).reshape(B, F, D)


# speedup vs baseline: 5.2378x; 1.6365x over previous
import functools

import jax
import jax.numpy as jnp
from jax import lax
from jax.experimental import pallas as pl
from jax.experimental.pallas import tpu as pltpu
from jax.experimental.pallas import tpu_sc as plsc

NC = 2
NS = 16
L = 16
NW = NC * NS

B = 16384
F = 26
D = 16
BW = B // NW
NBLK = BW // 128
TILE = 8 * 128
WTW = NBLK * TILE

TR = 1000000
CB = 512
NB1 = (TR // CB)
TAIL = TR - NB1 * CB
MAXI = NB1 // NW + 1


def _sc_repack(tabT, tailT):
    mesh = plsc.VectorSubcoreMesh(core_axis_name="c", subcore_axis_name="s")

    @functools.partial(
        pl.kernel,
        out_type=jax.ShapeDtypeStruct((TR // 8, 128), jnp.float32),
        mesh=mesh,
        compiler_params=pltpu.CompilerParams(
            use_tc_tiling_on_sc=True, needs_layout_passes=False
        ),
        scratch_types=[
            pltpu.VMEM((D, CB), jnp.float32),
            pltpu.VMEM((CB // 8, 128), jnp.float32),
            pltpu.VMEM((D, TAIL), jnp.float32),
        ],
    )
    def body(tabT_hbm, tail_hbm, out_hbm, in_v, out_v, tail_v):
        wid = lax.axis_index("s") * NC + lax.axis_index("c")
        lanes = lax.iota(jnp.int32, L)

        def transpose_cols(src_v, ncols):
            def grp(gg, carry):
                col = lanes + gg * L
                for j in range(D):
                    e_vec = (lanes + j) & (D - 1)
                    vec = plsc.load_gather(src_v, [e_vec, col])
                    flat = col * D + e_vec
                    plsc.store_scatter(out_v, [flat >> 7, flat & 127], vec)
                return carry

            lax.fori_loop(0, ncols // L, grp, 0)

        def step(i, carry):
            bidx = wid + NW * i

            @pl.when(bidx < NB1)
            def _():
                c0 = bidx * CB
                pltpu.sync_copy(tabT_hbm.at[:, pl.ds(c0, CB)], in_v)
                transpose_cols(in_v, CB)
                pltpu.sync_copy(out_v, out_hbm.at[pl.ds(bidx * (CB // 8), CB // 8)])

            return carry

        lax.fori_loop(0, MAXI, step, 0)

        @pl.when(wid == 1)
        def _tail():
            pltpu.sync_copy(tail_hbm, tail_v)
            transpose_cols(tail_v, TAIL)
            pltpu.sync_copy(
                out_v.at[pl.ds(0, TAIL * D // 128)],
                out_hbm.at[pl.ds(NB1 * (CB // 8), TAIL * D // 128)],
            )

    return body(tabT, tailT)


def _sc_embed(table, ids3, valT):
    mesh = plsc.VectorSubcoreMesh(core_axis_name="c", subcore_axis_name="s")

    @functools.partial(
        pl.kernel,
        out_type=jax.ShapeDtypeStruct((F, 2, (B // 128) * TILE), jnp.float32),
        mesh=mesh,
        compiler_params=pltpu.CompilerParams(
            use_tc_tiling_on_sc=False, needs_layout_passes=False
        ),
        scratch_types=[
            pltpu.VMEM((F, NBLK, 128), jnp.int32),
            pltpu.VMEM((F, BW), jnp.float32),
            pltpu.VMEM((2, BW, D), jnp.float32),
            pltpu.VMEM((2, 2, WTW), jnp.float32),
            pltpu.SemaphoreType.DMA,
            pltpu.SemaphoreType.DMA,
        ],
    )
    def body(tab_hbm, ids_hbm, val_hbm, out_hbm,
             idx_v, val_v, rows_v, stage_v, sem_g, sem_o):
        wid = lax.axis_index("s") * NC + lax.axis_index("c")
        b0 = wid * BW
        pltpu.sync_copy(ids_hbm.at[:, pl.ds(wid * NBLK, NBLK), :], idx_v)
        pltpu.sync_copy(val_hbm.at[:, pl.ds(b0, BW)], val_v)

        lanes = lax.iota(jnp.int32, L)

        def fire_gathers(f, p):
            return [
                pltpu.async_copy(
                    tab_hbm.at[idx_v.at[f, blk]],
                    rows_v.at[p, pl.ds(blk * 128, 128)],
                    sem_g,
                )
                for blk in range(NBLK)
            ]

        def compute_field(f, p):
            pconst = jnp.full((L,), p, jnp.int32)

            def grp(g, carry):
                valv = val_v[f, pl.ds(g * L, L)]
                ridx = lanes + g * L
                off = (g // 8) * TILE + (g % 8) * L
                for j in range(D):
                    e_vec = (lanes + j) & (D - 1)
                    vec = plsc.load_gather(rows_v, [pconst, ridx, e_vec])
                    plsc.store_scatter(
                        stage_v,
                        [pconst, e_vec >> 3, off + (e_vec & 7) * 128 + lanes],
                        vec * valv,
                    )
                return carry

            lax.fori_loop(0, BW // L, grp, 0)

        gacc = {0: fire_gathers(0, 0)}
        oacc = {}
        for f in range(F):
            p = f % 2
            if f + 1 < F:
                gacc[f + 1] = fire_gathers(f + 1, (f + 1) % 2)
            for cp in gacc[f]:
                cp.wait()
            compute_field(f, p)
            if f >= 2:
                for cp in oacc[f - 2]:
                    cp.wait()
            oacc[f] = [
                pltpu.async_copy(
                    stage_v.at[p, s],
                    out_hbm.at[f, s, pl.ds(wid * WTW, WTW)],
                    sem_o,
                )
                for s in range(2)
            ]
        for f in (F - 2, F - 1):
            for cp in oacc[f]:
                cp.wait()

    return body(table, ids3, valT)


def kernel(ids, value, table):
    ids3 = ids.T.astype(jnp.int32).reshape(F, B // 128, 128)
    valT = value.T
    tabT = table.T
    tab_lin = _sc_repack(tabT, tabT[:, NB1 * CB:]).reshape(TR, D)
    out = _sc_embed(tab_lin, ids3, valT)
    out5 = out.reshape(F, 2, B // 128, 8, 128)
    return out5.transpose(2, 4, 0, 1, 3).reshape(B, F, D)
